# baseline (speedup 1.0000x reference)
import functools

import jax
import jax.numpy as jnp
from jax import lax
from jax.experimental import pallas as pl
from jax.experimental.pallas import tpu as pltpu
from jax.experimental.pallas import tpu_sc as plsc

N = 4096
NFEAT = 512
NHID = 128
NCLASS = 64
HEADS = 4
MASK_DT = jnp.float8_e4m3fn
LOG2E = 1.4426950408889634

NW = 256
KMAX = 96
NWORK = 32
RPW = N // NWORK
NBUF = 4


def _proj_body(x_ref, w_ref, o0, o1, o2, o3):
    h = jnp.dot(x_ref[...], w_ref[...], preferred_element_type=jnp.float32)
    o0[...] = h[:, 0 * NHID:1 * NHID]
    o1[...] = h[:, 1 * NHID:2 * NHID]
    o2[...] = h[:, 2 * NHID:3 * NHID]
    o3[...] = h[:, 3 * NHID:4 * NHID]


def _proj(x, wcat):
    BM = 512
    out = jax.ShapeDtypeStruct((N, NHID), jnp.float32)
    return pl.pallas_call(
        _proj_body,
        grid=(N // BM,),
        in_specs=[
            pl.BlockSpec((BM, NFEAT), lambda i: (i, 0)),
            pl.BlockSpec((NFEAT, HEADS * NHID), lambda i: (0, 0)),
        ],
        out_specs=[pl.BlockSpec((BM, NHID), lambda i: (i, 0))] * HEADS,
        out_shape=[out] * HEADS,
    )(x, wcat)


def _pack_body(m_ref, pp_ref, o_ref):
    m1b = m_ref[...].astype(jnp.bfloat16)
    o_ref[...] = jnp.dot(m1b, pp_ref[...],
                         preferred_element_type=jnp.float32).astype(jnp.int32)


def _pack(m1, ppack):
    BM = 512
    return pl.pallas_call(
        _pack_body,
        grid=(N // BM,),
        in_specs=[
            pl.BlockSpec((BM, N), lambda i: (i, 0)),
            pl.BlockSpec((N, NW), lambda i: (0, 0)),
        ],
        out_specs=pl.BlockSpec((BM, NW), lambda i: (i, 0)),
        out_shape=jax.ShapeDtypeStruct((N, NW), jnp.int32),
    )(m1, ppack)


def _unpack_body(w_ref, e_ref, o_ref):
    w = w_ref[...]
    whi = (w >> 8).astype(jnp.bfloat16)
    wlo = (w & 255).astype(jnp.bfloat16)
    stacked = jnp.concatenate([whi, wlo], axis=1)
    rep = jnp.dot(stacked, e_ref[...],
                  preferred_element_type=jnp.float32)
    u = rep.astype(jnp.int32)
    o_ref[...] = (u & 1).astype(jnp.float32).astype(MASK_DT)


def _unpack(p, emat):
    BM = 512
    return pl.pallas_call(
        _unpack_body,
        grid=(N // BM,),
        in_specs=[
            pl.BlockSpec((BM, NW), lambda i: (i, 0)),
            pl.BlockSpec((2 * NW, N), lambda i: (0, 0)),
        ],
        out_specs=pl.BlockSpec((BM, N), lambda i: (i, 0)),
        out_shape=jax.ShapeDtypeStruct((N, N), MASK_DT),
    )(p, emat)


def _lane():
    return lax.iota(jnp.int32, 16)


def _or_pass(base, nbr_v, gbufs, stage_v, degv, table_hbm, out_hbm, sems):
    def row_body(rl, _):
        row_off = rl * KMAX
        dsplat = plsc.load_gather(degv, [jnp.full((16,), rl, jnp.int32)])
        nch = jnp.max(dsplat) // 8

        def fire(c):
            idx_ref = nbr_v.at[pl.ds(row_off + c * 8, 8)]
            b = lax.rem(c, NBUF)
            for k in range(NBUF):
                @pl.when(b == k)
                def _():
                    pltpu.async_copy(table_hbm.at[idx_ref], gbufs[k], sems[k])

        def prime(c, _):
            @pl.when(c < nch)
            def _():
                fire(c)
            return 0
        lax.fori_loop(0, NBUF, prime, 0)

        zero = jnp.zeros((16,), jnp.int32)
        srow = lax.rem(rl, 8) * NW
        for t in range(16):
            stage_v[pl.ds(srow + t * 16, 16)] = zero

        def chunk(c, _):
            b = lax.rem(c, NBUF)
            for k in range(NBUF):
                @pl.when(b == k)
                def _():
                    pltpu.make_async_copy(
                        table_hbm.at[nbr_v.at[pl.ds(row_off, 8)]],
                        gbufs[k], sems[k]).wait()
                    for t in range(16):
                        v = stage_v[pl.ds(srow + t * 16, 16)]
                        for rr in range(8):
                            v = v | gbufs[k][rr, pl.ds(t * 16, 16)]
                        stage_v[pl.ds(srow + t * 16, 16)] = v
            @pl.when(c + NBUF < nch)
            def _():
                fire(c + NBUF)
            return 0

        lax.fori_loop(0, nch, chunk, 0)

        @pl.when(lax.rem(rl, 8) == 7)
        def _():
            pltpu.sync_copy(
                stage_v,
                out_hbm.at[pl.ds(pl.multiple_of((base + rl - 7) * NW, 8 * NW),
                                 8 * NW)])
        return 0

    lax.fori_loop(0, RPW, row_body, 0)


def _mesh():
    return plsc.VectorSubcoreMesh(core_axis_name="c", subcore_axis_name="s")


def _make_hop1():
    @functools.partial(
        pl.kernel, mesh=_mesh(),
        out_type=[
            jax.ShapeDtypeStruct((N * NW,), jnp.int32),
            jax.ShapeDtypeStruct((N * KMAX,), jnp.int32),
            jax.ShapeDtypeStruct((N,), jnp.int32),
        ],
        scratch_types=[
            pltpu.VMEM((8, NW), jnp.int32),
            pltpu.VMEM((RPW * KMAX,), jnp.int32),
            pltpu.VMEM((8 * NW,), jnp.int32),
            pltpu.VMEM((RPW,), jnp.int32),
        ] + [pltpu.VMEM((8, NW), jnp.int32)] * NBUF
          + [pltpu.SemaphoreType.DMA] * NBUF,
        compiler_params=pltpu.CompilerParams(needs_layout_passes=False),
    )
    def hop1(p1_hbm, p2_hbm, nbr_hbm, deg_hbm,
             row_v, nbr_v, stage_v, degv, *rest):
        gbufs, sems = rest[:NBUF], rest[NBUF:]
        wid = lax.axis_index("s") * 2 + lax.axis_index("c")
        base = wid * RPW
        lane = _lane()

        def extract(rl, _):
            r = base + rl
            rr8 = lax.rem(rl, 8)

            @pl.when(rr8 == 0)
            def _():
                pltpu.sync_copy(
                    p1_hbm.at[pl.ds(pl.multiple_of(r, 8), 8)], row_v)
            row_off = rl * KMAX

            def group(g, deg):
                v = plsc.load_gather(
                    row_v, [jnp.full((16,), rr8, jnp.int32),
                            g * 16 + lane])
                nz = v != 0

                def cond(c):
                    return jnp.any(c[0])

                def body(c):
                    nz_, d = c
                    ffs = plsc.all_reduce_ffs(nz_)
                    widx = g * 16 + ffs
                    wv = plsc.load_gather(
                        row_v, [jnp.full((16,), rr8, jnp.int32), widx])
                    bm = ((wv >> lane) & 1) != 0
                    colv = widx * 16 + lane
                    cnt = jnp.sum(jnp.where(bm, 1, 0))
                    ok = d + 16 <= KMAX

                    @pl.when(ok)
                    def _():
                        plsc.store_compressed(
                            nbr_v.at[pl.ds(row_off + d, 16)], colv, mask=bm)
                    nz2 = nz_ & (lane != ffs)
                    return nz2, d + jnp.where(ok, cnt, 0)

                nz, deg = lax.while_loop(cond, body, (nz, deg))
                return deg

            deg = lax.fori_loop(0, 16, group, 0)
            padn = lax.rem(8 - lax.rem(deg, 8), 8)

            @pl.when(padn > 0)
            def _():
                plsc.store_compressed(
                    nbr_v.at[pl.ds(row_off + deg, 16)],
                    jnp.full((16,), r, jnp.int32), mask=lane < padn)
            plsc.store_scatter(degv, [jnp.full((16,), rl, jnp.int32)],
                               jnp.full((16,), deg + padn, jnp.int32),
                               mask=lane == 0)
            return 0

        lax.fori_loop(0, RPW, extract, 0)
        _or_pass(base, nbr_v, gbufs, stage_v, degv, p1_hbm, p2_hbm, sems)
        pltpu.sync_copy(nbr_v, nbr_hbm.at[pl.ds(base * KMAX, RPW * KMAX)])
        pltpu.sync_copy(degv, deg_hbm.at[pl.ds(base, RPW)])

    return hop1


def _make_hop2():
    @functools.partial(
        pl.kernel, mesh=_mesh(),
        out_type=jax.ShapeDtypeStruct((N * NW,), jnp.int32),
        scratch_types=[
            pltpu.VMEM((RPW * KMAX,), jnp.int32),
            pltpu.VMEM((8 * NW,), jnp.int32),
            pltpu.VMEM((RPW,), jnp.int32),
        ] + [pltpu.VMEM((8, NW), jnp.int32)] * NBUF
          + [pltpu.SemaphoreType.DMA] * NBUF,
        compiler_params=pltpu.CompilerParams(needs_layout_passes=False),
    )
    def hop2(p2_hbm, nbr_hbm, deg_hbm, p3_hbm, nbr_v, stage_v, degv, *rest):
        gbufs, sems = rest[:NBUF], rest[NBUF:]
        wid = lax.axis_index("s") * 2 + lax.axis_index("c")
        base = wid * RPW
        pltpu.sync_copy(nbr_hbm.at[pl.ds(base * KMAX, RPW * KMAX)], nbr_v)
        pltpu.sync_copy(deg_hbm.at[pl.ds(base, RPW)], degv)
        _or_pass(base, nbr_v, gbufs, stage_v, degv, p2_hbm, p3_hbm, sems)

    return hop2


def _gat_body(h_ref, hf_ref, a_ref, m_ref, o_ref):
    h = h_ref[...]
    hfull = hf_ref[...]
    a1 = a_ref[0:1, :] * LOG2E
    a2 = a_ref[1:2, :] * LOG2E
    f1 = jnp.sum(h * a1, axis=1, keepdims=True)
    f2 = jnp.sum(hfull * a2, axis=1, keepdims=True)
    s = f1 + f2.T
    t = jnp.minimum(jnp.maximum(s, 0.2 * s), 80.0)
    p = jnp.exp2(t) * m_ref[...].astype(jnp.float32)
    denom = jnp.sum(p, axis=1, keepdims=True)
    out = jnp.dot(p.astype(jnp.bfloat16), hfull.astype(jnp.bfloat16),
                  preferred_element_type=jnp.float32)
    o_ref[...] = out / denom


def _gat(h, a2d, mask):
    BM = 512
    return pl.pallas_call(
        _gat_body,
        grid=(N // BM,),
        in_specs=[
            pl.BlockSpec((BM, NHID), lambda i: (i, 0)),
            pl.BlockSpec((N, NHID), lambda i: (0, 0)),
            pl.BlockSpec((2, NHID), lambda i: (0, 0)),
            pl.BlockSpec((BM, N), lambda i: (i, 0)),
        ],
        out_specs=pl.BlockSpec((BM, NHID), lambda i: (i, 0)),
        out_shape=jax.ShapeDtypeStruct((N, NHID), jnp.float32),
    )(h, h, a2d, mask)


def _final_body(h0, h1, h2, h3, w_ref, b_ref, o_ref):
    h = jnp.concatenate(
        [jnp.maximum(h0[...], 0.0), jnp.maximum(h1[...], 0.0),
         jnp.maximum(h2[...], 0.0), jnp.maximum(h3[...], 0.0)], axis=1)
    logits = jnp.dot(h, w_ref[...], preferred_element_type=jnp.float32)
    logits = logits + b_ref[...]
    mx = jnp.max(logits, axis=1, keepdims=True)
    l = logits - mx
    lse = jnp.log(jnp.sum(jnp.exp(l), axis=1, keepdims=True))
    o_ref[...] = l - lse


def _final(parts, fc_wt, fc_b2d):
    BM = 512
    return pl.pallas_call(
        _final_body,
        grid=(N // BM,),
        in_specs=[pl.BlockSpec((BM, NHID), lambda i: (i, 0))] * HEADS + [
            pl.BlockSpec((HEADS * NHID, NCLASS), lambda i: (0, 0)),
            pl.BlockSpec((1, NCLASS), lambda i: (0, 0)),
        ],
        out_specs=pl.BlockSpec((BM, NCLASS), lambda i: (i, 0)),
        out_shape=jax.ShapeDtypeStruct((N, NCLASS), jnp.float32),
    )(*parts, fc_wt, fc_b2d)


def _packing_matrix():
    j = jnp.arange(N)
    g = jnp.arange(NW)
    return jnp.where(j[:, None] // 16 == g[None, :],
                     (2.0 ** (j % 16))[:, None].astype(jnp.float32),
                     0.0).astype(jnp.bfloat16)


def _expand_matrix():
    g = jnp.arange(NW)
    k = jnp.arange(N)
    sel = (g[:, None] == k[None, :] // 16).astype(jnp.float32)
    sc = (2.0 ** -(k % 16)).astype(jnp.float32)[None, :]
    return jnp.concatenate([sel * sc * 256.0, sel * sc],
                           axis=0).astype(jnp.bfloat16)


def kernel(x, adj, W, a, fc_w, fc_b):
    m1 = (adj > 0).astype(MASK_DT)
    p1 = _pack(m1, _packing_matrix())
    p2f, nbr, deg = _make_hop1()(p1)
    p2 = p2f.reshape(N, NW)
    p3 = _make_hop2()(p2, nbr, deg).reshape(N, NW)
    emat = _expand_matrix()
    m2 = _unpack(p2, emat)
    m3 = _unpack(p3, emat)

    wcat = jnp.concatenate([W[HEADS - 1], W[0], W[1], W[2]], axis=1)
    h3, h0, h1, h2 = _proj(x, wcat)

    masks = [m1, m2, m3]
    gouts = []
    for i, hh in enumerate([h0, h1, h2]):
        a2d = a[i].reshape(2, NHID)
        gouts.append(_gat(hh, a2d, masks[i]))

    return _final([h3] + gouts, fc_w.T, fc_b.reshape(1, NCLASS))


# speedup vs baseline: 2.1808x; 2.1808x over previous
import jax
import jax.numpy as jnp
from jax.experimental import pallas as pl

N = 4096
NFEAT = 512
NHID = 128
NCLASS = 64
HEADS = 4
MASK_DT = jnp.float8_e4m3fn
LOG2E = 1.4426950408889634


def _proj_body(x_ref, w_ref, o0, o1, o2, o3):
    h = jnp.dot(x_ref[...], w_ref[...], preferred_element_type=jnp.float32)
    o0[...] = h[:, 0 * NHID:1 * NHID]
    o1[...] = h[:, 1 * NHID:2 * NHID]
    o2[...] = h[:, 2 * NHID:3 * NHID]
    o3[...] = h[:, 3 * NHID:4 * NHID]


def _proj(x, wcat):
    BM = 512
    out = jax.ShapeDtypeStruct((N, NHID), jnp.float32)
    return pl.pallas_call(
        _proj_body,
        grid=(N // BM,),
        in_specs=[
            pl.BlockSpec((BM, NFEAT), lambda i: (i, 0)),
            pl.BlockSpec((NFEAT, HEADS * NHID), lambda i: (0, 0)),
        ],
        out_specs=[pl.BlockSpec((BM, NHID), lambda i: (i, 0))] * HEADS,
        out_shape=[out] * HEADS,
    )(x, wcat)


def _maskmm_body(a_ref, b_ref, o_ref):
    acc = jnp.dot(a_ref[...], b_ref[...], preferred_element_type=jnp.float32)
    o_ref[...] = (acc > 0).astype(MASK_DT)


def _maskmm(a, b):
    BM, BN = 512, 2048
    return pl.pallas_call(
        _maskmm_body,
        grid=(N // BN, N // BM),
        in_specs=[
            pl.BlockSpec((BM, N), lambda j, i: (i, 0)),
            pl.BlockSpec((N, BN), lambda j, i: (0, j)),
        ],
        out_specs=pl.BlockSpec((BM, BN), lambda j, i: (i, j)),
        out_shape=jax.ShapeDtypeStruct((N, N), MASK_DT),
    )(a, b)


def _gat3_body(h0b, h1b, h2b, hf0, hf1, hf2, a_ref,
               m0_ref, m1_ref, m2_ref, o0, o1, o2):
    for k, (hb, hf, m_ref, o_ref) in enumerate(
            [(h0b, hf0, m0_ref, o0), (h1b, hf1, m1_ref, o1),
             (h2b, hf2, m2_ref, o2)]):
        h = hb[...]
        hfull = hf[...]
        a1 = a_ref[2 * k:2 * k + 1, :] * LOG2E
        a2 = a_ref[2 * k + 1:2 * k + 2, :] * LOG2E
        f1 = jnp.sum(h * a1, axis=1, keepdims=True)
        f2 = jnp.sum(hfull * a2, axis=1, keepdims=True)
        s = f1 + f2.T
        t = jnp.minimum(jnp.maximum(s, 0.2 * s), 80.0)
        p = jnp.exp2(t) * m_ref[...].astype(jnp.float32)
        denom = jnp.sum(p, axis=1, keepdims=True)
        out = jnp.dot(p.astype(jnp.bfloat16), hfull.astype(jnp.bfloat16),
                      preferred_element_type=jnp.float32)
        o_ref[...] = out / denom


def _gat3(hs, a6, masks):
    BM = 512
    blk = lambda i: (i, 0)
    full = lambda i: (0, 0)
    return pl.pallas_call(
        _gat3_body,
        grid=(N // BM,),
        in_specs=[pl.BlockSpec((BM, NHID), blk)] * 3 +
                 [pl.BlockSpec((N, NHID), full)] * 3 +
                 [pl.BlockSpec((2 * (HEADS - 1), NHID), full)] +
                 [pl.BlockSpec((BM, N), blk)] * 3,
        out_specs=[pl.BlockSpec((BM, NHID), blk)] * 3,
        out_shape=[jax.ShapeDtypeStruct((N, NHID), jnp.float32)] * 3,
    )(*hs, *hs, a6, *masks)


def _final_body(h0, h1, h2, h3, w_ref, b_ref, o_ref):
    h = jnp.concatenate(
        [jnp.maximum(h0[...], 0.0), jnp.maximum(h1[...], 0.0),
         jnp.maximum(h2[...], 0.0), jnp.maximum(h3[...], 0.0)], axis=1)
    logits = jnp.dot(h, w_ref[...], preferred_element_type=jnp.float32)
    logits = logits + b_ref[...]
    mx = jnp.max(logits, axis=1, keepdims=True)
    l = logits - mx
    lse = jnp.log(jnp.sum(jnp.exp(l), axis=1, keepdims=True))
    o_ref[...] = l - lse


def _final(parts, fc_wt, fc_b2d):
    BM = 512
    return pl.pallas_call(
        _final_body,
        grid=(N // BM,),
        in_specs=[pl.BlockSpec((BM, NHID), lambda i: (i, 0))] * HEADS + [
            pl.BlockSpec((HEADS * NHID, NCLASS), lambda i: (0, 0)),
            pl.BlockSpec((1, NCLASS), lambda i: (0, 0)),
        ],
        out_specs=pl.BlockSpec((BM, NCLASS), lambda i: (i, 0)),
        out_shape=jax.ShapeDtypeStruct((N, NCLASS), jnp.float32),
    )(*parts, fc_wt, fc_b2d)


def kernel(x, adj, W, a, fc_w, fc_b):
    m1 = (adj > 0).astype(MASK_DT)
    m2 = _maskmm(m1, m1)
    m3 = _maskmm(m2, m1)

    wcat = jnp.concatenate([W[HEADS - 1], W[0], W[1], W[2]], axis=1)
    h3, h0, h1, h2 = _proj(x, wcat)

    a6 = a.reshape(2 * (HEADS - 1), NHID)
    gouts = list(_gat3([h0, h1, h2], a6, [m1, m2, m3]))

    return _final([h3] + gouts, fc_w.T, fc_b.reshape(1, NCLASS))


# speedup vs baseline: 2.2323x; 1.0236x over previous
import jax
import jax.numpy as jnp
from jax.experimental import pallas as pl

N = 4096
NFEAT = 512
NHID = 128
NCLASS = 64
HEADS = 4
MASK_DT = jnp.float8_e4m3fn
LOG2E = 1.4426950408889634


def _proj_body(x_ref, w_ref, o0, o1, o2, o3):
    h = jnp.dot(x_ref[...], w_ref[...], preferred_element_type=jnp.float32)
    o0[...] = h[:, 0 * NHID:1 * NHID]
    o1[...] = h[:, 1 * NHID:2 * NHID]
    o2[...] = h[:, 2 * NHID:3 * NHID]
    o3[...] = h[:, 3 * NHID:4 * NHID]


def _proj(x, wcat):
    BM = 512
    out = jax.ShapeDtypeStruct((N, NHID), jnp.float32)
    return pl.pallas_call(
        _proj_body,
        grid=(N // BM,),
        in_specs=[
            pl.BlockSpec((BM, NFEAT), lambda i: (i, 0)),
            pl.BlockSpec((NFEAT, HEADS * NHID), lambda i: (0, 0)),
        ],
        out_specs=[pl.BlockSpec((BM, NHID), lambda i: (i, 0))] * HEADS,
        out_shape=[out] * HEADS,
    )(x, wcat)


def _maskmm_body(a_ref, b_ref, o_ref):
    acc = jnp.dot(a_ref[...], b_ref[...], preferred_element_type=jnp.float32)
    o_ref[...] = (acc > 0).astype(MASK_DT)


def _maskmm(a, b):
    BM, BN = 512, 2048
    return pl.pallas_call(
        _maskmm_body,
        grid=(N // BN, N // BM),
        in_specs=[
            pl.BlockSpec((BM, N), lambda j, i: (i, 0)),
            pl.BlockSpec((N, BN), lambda j, i: (0, j)),
        ],
        out_specs=pl.BlockSpec((BM, BN), lambda j, i: (i, j)),
        out_shape=jax.ShapeDtypeStruct((N, N), MASK_DT),
    )(a, b)


def _gat3_body(h3b, h0b, h1b, h2b, hf0, hf1, hf2, a_ref,
               m0_ref, m1_ref, m2_ref, w_ref, b_ref, o_ref):
    w = w_ref[...]
    acc = jnp.dot(jnp.maximum(h3b[...], 0.0), w[0:NHID, :],
                  preferred_element_type=jnp.float32)
    for k, (hb, hf, m_ref) in enumerate(
            [(h0b, hf0, m0_ref), (h1b, hf1, m1_ref),
             (h2b, hf2, m2_ref)]):
        h = hb[...]
        hfull = hf[...]
        a1 = a_ref[2 * k:2 * k + 1, :] * LOG2E
        a2 = a_ref[2 * k + 1:2 * k + 2, :] * LOG2E
        f1 = jnp.sum(h * a1, axis=1, keepdims=True)
        f2 = jnp.sum(hfull * a2, axis=1, keepdims=True)
        s = f1 + f2.T
        t = jnp.minimum(jnp.maximum(s, 0.2 * s), 80.0)
        p = jnp.exp2(t) * m_ref[...].astype(jnp.float32)
        denom = jnp.sum(p, axis=1, keepdims=True)
        out = jnp.dot(p.astype(jnp.bfloat16), hfull.astype(jnp.bfloat16),
                      preferred_element_type=jnp.float32)
        g = out / denom
        acc += jnp.dot(jnp.maximum(g, 0.0),
                       w[NHID * (k + 1):NHID * (k + 2), :],
                       preferred_element_type=jnp.float32)
    logits = acc + b_ref[...]
    mx = jnp.max(logits, axis=1, keepdims=True)
    l = logits - mx
    lse = jnp.log(jnp.sum(jnp.exp(l), axis=1, keepdims=True))
    o_ref[...] = l - lse


def _gat3final(h3, hs, a6, masks, fc_wt, fc_b2d):
    BM = 512
    blk = lambda i: (i, 0)
    full = lambda i: (0, 0)
    return pl.pallas_call(
        _gat3_body,
        grid=(N // BM,),
        in_specs=[pl.BlockSpec((BM, NHID), blk)] * 4 +
                 [pl.BlockSpec((N, NHID), full)] * 3 +
                 [pl.BlockSpec((2 * (HEADS - 1), NHID), full)] +
                 [pl.BlockSpec((BM, N), blk)] * 3 + [
            pl.BlockSpec((HEADS * NHID, NCLASS), full),
            pl.BlockSpec((1, NCLASS), full),
        ],
        out_specs=pl.BlockSpec((BM, NCLASS), blk),
        out_shape=jax.ShapeDtypeStruct((N, NCLASS), jnp.float32),
    )(h3, *hs, *hs, a6, *masks, fc_wt, fc_b2d)


def _final_body(h0, h1, h2, h3, w_ref, b_ref, o_ref):
    h = jnp.concatenate(
        [jnp.maximum(h0[...], 0.0), jnp.maximum(h1[...], 0.0),
         jnp.maximum(h2[...], 0.0), jnp.maximum(h3[...], 0.0)], axis=1)
    logits = jnp.dot(h, w_ref[...], preferred_element_type=jnp.float32)
    logits = logits + b_ref[...]
    mx = jnp.max(logits, axis=1, keepdims=True)
    l = logits - mx
    lse = jnp.log(jnp.sum(jnp.exp(l), axis=1, keepdims=True))
    o_ref[...] = l - lse


def _final(parts, fc_wt, fc_b2d):
    BM = 512
    return pl.pallas_call(
        _final_body,
        grid=(N // BM,),
        in_specs=[pl.BlockSpec((BM, NHID), lambda i: (i, 0))] * HEADS + [
            pl.BlockSpec((HEADS * NHID, NCLASS), lambda i: (0, 0)),
            pl.BlockSpec((1, NCLASS), lambda i: (0, 0)),
        ],
        out_specs=pl.BlockSpec((BM, NCLASS), lambda i: (i, 0)),
        out_shape=jax.ShapeDtypeStruct((N, NCLASS), jnp.float32),
    )(*parts, fc_wt, fc_b2d)


def kernel(x, adj, W, a, fc_w, fc_b):
    m1 = (adj > 0).astype(MASK_DT)
    m2 = _maskmm(m1, m1)
    m3 = _maskmm(m2, m1)

    wcat = jnp.concatenate([W[HEADS - 1], W[0], W[1], W[2]], axis=1)
    h3, h0, h1, h2 = _proj(x, wcat)

    a6 = a.reshape(2 * (HEADS - 1), NHID)
    return _gat3final(h3, [h0, h1, h2], a6, [m1, m2, m3],
                      fc_w.T, fc_b.reshape(1, NCLASS))


# speedup vs baseline: 2.2541x; 1.0098x over previous
import jax
import jax.numpy as jnp
from jax.experimental import pallas as pl

N = 4096
NFEAT = 512
NHID = 128
NCLASS = 64
HEADS = 4
MASK_DT = jnp.float8_e4m3fn
LOG2E = 1.4426950408889634


def _proj_body(x_ref, w_ref, adj_ref, o0, o1, o2, o3, m1_ref):
    h = jnp.dot(x_ref[...], w_ref[...], preferred_element_type=jnp.float32)
    o0[...] = h[:, 0 * NHID:1 * NHID]
    o1[...] = h[:, 1 * NHID:2 * NHID]
    o2[...] = h[:, 2 * NHID:3 * NHID]
    o3[...] = h[:, 3 * NHID:4 * NHID]
    m1_ref[...] = (adj_ref[...] > 0).astype(MASK_DT)


def _proj(x, wcat, adj):
    BM = 512
    out = jax.ShapeDtypeStruct((N, NHID), jnp.float32)
    return pl.pallas_call(
        _proj_body,
        grid=(N // BM,),
        in_specs=[
            pl.BlockSpec((BM, NFEAT), lambda i: (i, 0)),
            pl.BlockSpec((NFEAT, HEADS * NHID), lambda i: (0, 0)),
            pl.BlockSpec((BM, N), lambda i: (i, 0)),
        ],
        out_specs=[pl.BlockSpec((BM, NHID), lambda i: (i, 0))] * HEADS +
                  [pl.BlockSpec((BM, N), lambda i: (i, 0))],
        out_shape=[out] * HEADS + [jax.ShapeDtypeStruct((N, N), MASK_DT)],
    )(x, wcat, adj)


def _maskmm_body(a_ref, b_ref, o_ref):
    acc = jnp.dot(a_ref[...], b_ref[...], preferred_element_type=jnp.float32)
    o_ref[...] = (acc > 0).astype(MASK_DT)


def _maskmm(a, b):
    BM, BN = 512, 2048
    return pl.pallas_call(
        _maskmm_body,
        grid=(N // BN, N // BM),
        in_specs=[
            pl.BlockSpec((BM, N), lambda j, i: (i, 0)),
            pl.BlockSpec((N, BN), lambda j, i: (0, j)),
        ],
        out_specs=pl.BlockSpec((BM, BN), lambda j, i: (i, j)),
        out_shape=jax.ShapeDtypeStruct((N, N), MASK_DT),
    )(a, b)


def _gat3_body(h3b, h0b, h1b, h2b, hf0, hf1, hf2, a_ref,
               m0_ref, m1_ref, m2_ref, w_ref, b_ref, o_ref):
    w = w_ref[...]
    acc = jnp.dot(jnp.maximum(h3b[...], 0.0), w[0:NHID, :],
                  preferred_element_type=jnp.float32)
    for k, (hb, hf, m_ref) in enumerate(
            [(h0b, hf0, m0_ref), (h1b, hf1, m1_ref),
             (h2b, hf2, m2_ref)]):
        h = hb[...]
        hfull = hf[...]
        a1 = a_ref[2 * k:2 * k + 1, :] * LOG2E
        a2 = a_ref[2 * k + 1:2 * k + 2, :] * LOG2E
        f1 = jnp.sum(h * a1, axis=1, keepdims=True)
        f2 = jnp.sum(hfull * a2, axis=1, keepdims=True)
        s = f1 + f2.T
        t = jnp.minimum(jnp.maximum(s, 0.2 * s), 80.0)
        p = jnp.exp2(t) * m_ref[...].astype(jnp.float32)
        denom = jnp.sum(p, axis=1, keepdims=True)
        out = jnp.dot(p.astype(jnp.bfloat16), hfull.astype(jnp.bfloat16),
                      preferred_element_type=jnp.float32)
        g = out / denom
        acc += jnp.dot(jnp.maximum(g, 0.0),
                       w[NHID * (k + 1):NHID * (k + 2), :],
                       preferred_element_type=jnp.float32)
    logits = acc + b_ref[...]
    mx = jnp.max(logits, axis=1, keepdims=True)
    l = logits - mx
    lse = jnp.log(jnp.sum(jnp.exp(l), axis=1, keepdims=True))
    o_ref[...] = l - lse


def _gat3final(h3, hs, a6, masks, fc_wt, fc_b2d):
    BM = 512
    blk = lambda i: (i, 0)
    full = lambda i: (0, 0)
    return pl.pallas_call(
        _gat3_body,
        grid=(N // BM,),
        in_specs=[pl.BlockSpec((BM, NHID), blk)] * 4 +
                 [pl.BlockSpec((N, NHID), full)] * 3 +
                 [pl.BlockSpec((2 * (HEADS - 1), NHID), full)] +
                 [pl.BlockSpec((BM, N), blk)] * 3 + [
            pl.BlockSpec((HEADS * NHID, NCLASS), full),
            pl.BlockSpec((1, NCLASS), full),
        ],
        out_specs=pl.BlockSpec((BM, NCLASS), blk),
        out_shape=jax.ShapeDtypeStruct((N, NCLASS), jnp.float32),
    )(h3, *hs, *hs, a6, *masks, fc_wt, fc_b2d)


def _final_body(h0, h1, h2, h3, w_ref, b_ref, o_ref):
    h = jnp.concatenate(
        [jnp.maximum(h0[...], 0.0), jnp.maximum(h1[...], 0.0),
         jnp.maximum(h2[...], 0.0), jnp.maximum(h3[...], 0.0)], axis=1)
    logits = jnp.dot(h, w_ref[...], preferred_element_type=jnp.float32)
    logits = logits + b_ref[...]
    mx = jnp.max(logits, axis=1, keepdims=True)
    l = logits - mx
    lse = jnp.log(jnp.sum(jnp.exp(l), axis=1, keepdims=True))
    o_ref[...] = l - lse


def _final(parts, fc_wt, fc_b2d):
    BM = 512
    return pl.pallas_call(
        _final_body,
        grid=(N // BM,),
        in_specs=[pl.BlockSpec((BM, NHID), lambda i: (i, 0))] * HEADS + [
            pl.BlockSpec((HEADS * NHID, NCLASS), lambda i: (0, 0)),
            pl.BlockSpec((1, NCLASS), lambda i: (0, 0)),
        ],
        out_specs=pl.BlockSpec((BM, NCLASS), lambda i: (i, 0)),
        out_shape=jax.ShapeDtypeStruct((N, NCLASS), jnp.float32),
    )(*parts, fc_wt, fc_b2d)


def kernel(x, adj, W, a, fc_w, fc_b):
    wcat = jnp.concatenate([W[HEADS - 1], W[0], W[1], W[2]], axis=1)
    h3, h0, h1, h2, m1 = _proj(x, wcat, adj)
    m2 = _maskmm(m1, m1)
    m3 = _maskmm(m2, m1)

    a6 = a.reshape(2 * (HEADS - 1), NHID)
    return _gat3final(h3, [h0, h1, h2], a6, [m1, m2, m3],
                      fc_w.T, fc_b.reshape(1, NCLASS))
